# Initial kernel scaffold; baseline (speedup 1.0000x reference)
import jax
import jax.numpy as jnp
from jax.experimental import pallas as pl


def kernel(expert_activations, attractor_centers, attraction_radii):
    raise NotImplementedError("write your pallas kernel here")


# speedup vs baseline: 1.2725x; 1.2725x over previous
import jax
import jax.numpy as jnp
from jax.experimental import pallas as pl

BATCH = 16384
E = 64
BT = 256


def _body(x_ref, c_ref, r_ref, out_ref, idx_ref):
    x = x_ref[...]
    c = c_ref[...]
    r = r_ref[...]

    diff = x[:, None, :] - c[None, :, :]
    d2 = jnp.sum(diff * diff, axis=-1)
    best = jnp.argmin(d2, axis=1)
    mind = jnp.sqrt(jnp.min(d2, axis=1))

    onehot = (jax.lax.broadcasted_iota(jnp.int32, (BT, E), 1)
              == best[:, None]).astype(jnp.float32)
    rsel = jnp.sum(onehot * r, axis=1)
    s = 0.1 * jnp.exp(-mind / (rsel + 1e-8))
    csel = jnp.dot(onehot, c, preferred_element_type=jnp.float32)

    out_ref[...] = x * (1.0 - s)[:, None] + csel * s[:, None]
    idx_ref[...] = best[:, None].astype(jnp.int32)


def kernel(expert_activations, attractor_centers, attraction_radii):
    radii2d = attraction_radii.reshape(1, E)
    attracted, closest = pl.pallas_call(
        _body,
        grid=(BATCH // BT,),
        in_specs=[
            pl.BlockSpec((BT, E), lambda i: (i, 0)),
            pl.BlockSpec((E, E), lambda i: (0, 0)),
            pl.BlockSpec((1, E), lambda i: (0, 0)),
        ],
        out_specs=[
            pl.BlockSpec((BT, E), lambda i: (i, 0)),
            pl.BlockSpec((BT, 1), lambda i: (i, 0)),
        ],
        out_shape=[
            jax.ShapeDtypeStruct((BATCH, E), jnp.float32),
            jax.ShapeDtypeStruct((BATCH, 1), jnp.int32),
        ],
    )(expert_activations, attractor_centers, radii2d)
    return attracted, closest.reshape(BATCH)
